# baseline (speedup 1.0000x reference)
import jax
import jax.numpy as jnp
from jax import lax
from jax.experimental import pallas as pl
from jax.experimental.pallas import tpu as pltpu
from jax.experimental.pallas import tpu_sc as plsc

NC = 2
NS = 16
LANES = 16


def _split_dot(v, m):
    hi = v.astype(jnp.bfloat16).astype(jnp.float32)
    return (jnp.dot(hi, m, preferred_element_type=jnp.float32)
            + jnp.dot(v - hi, m, preferred_element_type=jnp.float32))


def _gather_dot(onehot, tbl):
    hi = tbl.astype(jnp.bfloat16).astype(jnp.float32)
    return (jnp.dot(onehot, hi, preferred_element_type=jnp.float32)
            + jnp.dot(onehot, tbl - hi, preferred_element_type=jnp.float32))


def _edge_body(ebi_ref, ea_ref, instr_ref, wedge_ref, wrel_ref, out_ref):
    be = ea_ref.shape[0]
    b = instr_ref.shape[0]
    onehot = (ebi_ref[...] == lax.broadcasted_iota(jnp.int32, (be, b), 1))
    instr_g = jnp.dot(onehot.astype(jnp.float32), instr_ref[...],
                      preferred_element_type=jnp.float32)
    y = jnp.dot(instr_g * ea_ref[...], wedge_ref[...],
                preferred_element_type=jnp.float32)
    y = jnp.where(y > 0, y, jnp.exp(y) - 1.0)
    out_ref[...] = jnp.sum(y * wrel_ref[...], axis=1, keepdims=True)


def _edge_scores(edge_attrs, instruction_batch, w_edge, w_rel, ebi):
    e, h = edge_attrs.shape
    b = instruction_batch.shape[0]
    be = 3200
    return pl.pallas_call(
        _edge_body,
        grid=(e // be,),
        in_specs=[
            pl.BlockSpec((be, 1), lambda i: (i, 0)),
            pl.BlockSpec((be, h), lambda i: (i, 0)),
            pl.BlockSpec((b, h), lambda i: (0, 0)),
            pl.BlockSpec((h, h), lambda i: (0, 0)),
            pl.BlockSpec((1, h), lambda i: (0, 0)),
        ],
        out_specs=pl.BlockSpec((be, 1), lambda i: (i, 0)),
        out_shape=jax.ShapeDtypeStruct((e, 1), jnp.float32),
    )(ebi.reshape(e, 1), edge_attrs, instruction_batch, w_edge,
      w_rel.reshape(1, h))


def _node_body(ni_ref, na_ref, instr_ref, nps_ref, wprops_ref, wnode_ref,
               out_ref):
    bn = na_ref.shape[0]
    b, h = instr_ref.shape
    p_cnt = wprops_ref.shape[0]
    onehot = (ni_ref[...] == lax.broadcasted_iota(jnp.int32, (bn, b), 1))
    onehot = onehot.astype(jnp.float32)
    ib = _gather_dot(onehot, instr_ref[...])
    aps = _gather_dot(onehot, nps_ref[...])
    acc = jnp.zeros((bn, h), jnp.float32)
    for p in range(p_cnt):
        zp = aps[:, p:p + 1] * ib * na_ref[:, p, :]
        acc = acc + jnp.dot(zp, wprops_ref[p],
                            preferred_element_type=jnp.float32)
    y = jnp.where(acc > 0, acc, jnp.exp(acc) - 1.0)
    out_ref[...] = jnp.sum(y * wnode_ref[...], axis=1, keepdims=True)


def _node_scores(node_attrs, instruction_batch, node_prop_similarities,
                 w_props, w_node, ni):
    n, p_cnt, h = node_attrs.shape
    b = instruction_batch.shape[0]
    bn = 1000
    return pl.pallas_call(
        _node_body,
        grid=(n // bn,),
        in_specs=[
            pl.BlockSpec((bn, 1), lambda i: (i, 0)),
            pl.BlockSpec((bn, p_cnt, h), lambda i: (i, 0, 0)),
            pl.BlockSpec((b, h), lambda i: (0, 0)),
            pl.BlockSpec((b, p_cnt), lambda i: (0, 0)),
            pl.BlockSpec((p_cnt, h, h), lambda i: (0, 0, 0)),
            pl.BlockSpec((1, h), lambda i: (0, 0)),
        ],
        out_specs=pl.BlockSpec((bn, 1), lambda i: (i, 0)),
        out_shape=jax.ShapeDtypeStruct((n, 1), jnp.float32),
    )(ni.reshape(n, 1), node_attrs, instruction_batch,
      node_prop_similarities, w_props, w_node.reshape(1, h))


def _scores_body(ebi_ref, ea_ref, ni_ref, na_ref, instr_ref, nps_ref,
                 wedge_ref, wprops_ref, wrel_ref, wnode_ref,
                 s_ref, x_ref):
    be = ea_ref.shape[0]
    b, h = instr_ref.shape
    p_cnt = wprops_ref.shape[0]
    bn = na_ref.shape[0]

    onehot_e = (ebi_ref[...] == lax.broadcasted_iota(jnp.int32, (be, b), 1))
    instr_g = jnp.dot(onehot_e.astype(jnp.float32), instr_ref[...],
                      preferred_element_type=jnp.float32)
    y = jnp.dot(instr_g * ea_ref[...], wedge_ref[...],
                preferred_element_type=jnp.float32)
    y = jnp.where(y > 0, y, jnp.exp(y) - 1.0)
    s_ref[...] = jnp.sum(y * wrel_ref[...], axis=1, keepdims=True)

    @pl.when(pl.program_id(0) % NODE_EVERY == 0)
    def _node():
        onehot = (ni_ref[...] == lax.broadcasted_iota(jnp.int32, (bn, b), 1))
        onehot = onehot.astype(jnp.float32)
        ib = _gather_dot(onehot, instr_ref[...])
        aps = _gather_dot(onehot, nps_ref[...])
        acc = jnp.zeros((bn, h), jnp.float32)
        for p in range(p_cnt):
            zp = aps[:, p:p + 1] * ib * na_ref[:, p, :]
            acc = acc + jnp.dot(zp, wprops_ref[p],
                                preferred_element_type=jnp.float32)
        yk = jnp.where(acc > 0, acc, jnp.exp(acc) - 1.0)
        x_ref[...] = jnp.sum(yk * wnode_ref[...], axis=1, keepdims=True)


NODE_EVERY = 10


def _all_scores(edge_attrs, node_attrs, instruction_batch,
                node_prop_similarities, w_edge, w_props, w_rel, w_node,
                ebi, ni):
    e, h = edge_attrs.shape
    n, p_cnt, _ = node_attrs.shape
    b = instruction_batch.shape[0]
    be = 3200
    grid = e // be
    bn = n // (grid // NODE_EVERY)
    return pl.pallas_call(
        _scores_body,
        grid=(grid,),
        in_specs=[
            pl.BlockSpec((be, 1), lambda i: (i, 0)),
            pl.BlockSpec((be, h), lambda i: (i, 0)),
            pl.BlockSpec((bn, 1), lambda i: (i // NODE_EVERY, 0)),
            pl.BlockSpec((bn, p_cnt, h), lambda i: (i // NODE_EVERY, 0, 0)),
            pl.BlockSpec((b, h), lambda i: (0, 0)),
            pl.BlockSpec((b, p_cnt), lambda i: (0, 0)),
            pl.BlockSpec((h, h), lambda i: (0, 0)),
            pl.BlockSpec((p_cnt, h, h), lambda i: (0, 0, 0)),
            pl.BlockSpec((1, h), lambda i: (0, 0)),
            pl.BlockSpec((1, h), lambda i: (0, 0)),
        ],
        out_specs=[
            pl.BlockSpec((be, 1), lambda i: (i, 0)),
            pl.BlockSpec((bn, 1), lambda i: (i // NODE_EVERY, 0)),
        ],
        out_shape=[
            jax.ShapeDtypeStruct((e, 1), jnp.float32),
            jax.ShapeDtypeStruct((n, 1), jnp.float32),
        ],
    )(ebi.reshape(e, 1), edge_attrs, ni.reshape(n, 1), node_attrs,
      instruction_batch, node_prop_similarities, w_edge, w_props,
      w_rel.reshape(1, h), w_node.reshape(1, h))


def _sc_scatter_body(n_pad, ec, s_hbm, src_hbm, dst_hbm, dist_hbm, zeros_hbm,
                     part_hbm, s_v, src_v, dst_v, dist_v, acc_v, tmp_v, red_v,
                     kbuf_v, shared_acc):
    c = lax.axis_index("c")
    s = lax.axis_index("s")
    wid = c * NS + s
    ebase = wid * ec
    nslice = n_pad // NS

    pltpu.sync_copy(s_hbm.at[pl.ds(ebase, ec)], s_v)
    pltpu.sync_copy(src_hbm.at[pl.ds(ebase, ec)], src_v)
    pltpu.sync_copy(dst_hbm.at[pl.ds(ebase, ec)], dst_v)
    pltpu.sync_copy(dist_hbm, dist_v)
    pltpu.sync_copy(zeros_hbm, acc_v)

    lanes_iota = lax.iota(jnp.int32, LANES)
    kbuf_v[pl.ds(LANES, LANES)] = jnp.full((LANES,), -1, jnp.int32)

    @pl.loop(0, ec // LANES, unroll=4)
    def _edges(i):
        sl = pl.ds(i * LANES, LANES)
        d16 = plsc.load_gather(dist_v, [src_v[sl]])
        k, v = plsc.sort_key_val(dst_v[sl], d16 * s_v[sl])
        cs = plsc.cumsum(v)
        kbuf_v[pl.ds(0, LANES)] = k
        k_next = plsc.load_gather(kbuf_v, [lanes_iota + 1])
        last = k != k_next
        plsc.addupdate_scatter(acc_v, [k], cs, mask=last)
        plsc.addupdate_scatter(acc_v, [k_next], -cs,
                               mask=last & (lanes_iota < LANES - 1))

    pltpu.sync_copy(acc_v, shared_acc.at[s])
    plsc.subcore_barrier()
    nbase = s * nslice
    pltpu.sync_copy(zeros_hbm.at[pl.ds(0, nslice)], red_v)

    @pl.loop(0, NS)
    def _tiles(t):
        pltpu.sync_copy(shared_acc.at[t, pl.ds(nbase, nslice)], tmp_v)

        @pl.loop(0, nslice // LANES)
        def _acc(j):
            sl = pl.ds(j * LANES, LANES)
            red_v[sl] = red_v[sl] + tmp_v[sl]

    pltpu.sync_copy(red_v, part_hbm.at[c, pl.ds(nbase, nslice)])


def _sc_scatter(s_flat, src, dst, dist, zeros, n_pad):
    e = s_flat.shape[0]
    n = dist.shape[0]
    ec = e // (NC * NS)
    nslice = n_pad // NS
    mesh = plsc.VectorSubcoreMesh(core_axis_name="c", subcore_axis_name="s",
                                  num_cores=NC, num_subcores=NS)
    import functools
    body = functools.partial(_sc_scatter_body, n_pad, ec)
    return pl.kernel(
        body,
        out_type=jax.ShapeDtypeStruct((NC, n_pad), jnp.float32),
        mesh=mesh,
        compiler_params=pltpu.CompilerParams(needs_layout_passes=False),
        scratch_types=[
            pltpu.VMEM((ec,), jnp.float32),
            pltpu.VMEM((ec,), jnp.int32),
            pltpu.VMEM((ec,), jnp.int32),
            pltpu.VMEM((n,), jnp.float32),
            pltpu.VMEM((n_pad,), jnp.float32),
            pltpu.VMEM((nslice,), jnp.float32),
            pltpu.VMEM((nslice,), jnp.float32),
            pltpu.VMEM((2 * LANES,), jnp.int32),
            pltpu.VMEM_SHARED((NS, n_pad), jnp.float32),
        ],
    )(s_flat, src, dst, dist, zeros)


def _final_body(part_ref, xs_ref, nic_ref, nir_ref, rel_ref, out_ref):
    n_pad = part_ref.shape[1]
    b = rel_ref.shape[1]
    red = part_ref[0:1, :] + part_ref[1:2, :]
    er = jnp.exp(red)
    es = jnp.exp(xs_ref[...])
    onehot = (nic_ref[...] == lax.broadcasted_iota(jnp.int32, (n_pad, b), 1))
    onehot = onehot.astype(jnp.float32)
    onehot_t = (nir_ref[...] == lax.broadcasted_iota(jnp.int32, (b, n_pad), 0))
    onehot_t = onehot_t.astype(jnp.float32)
    s_r = _split_dot(er, onehot)
    s_s = _split_dot(es, onehot)
    srn = jnp.maximum(_split_dot(s_r, onehot_t), 1e-30)
    ssn = jnp.maximum(_split_dot(s_s, onehot_t), 1e-30)
    rsn = _split_dot(rel_ref[...], onehot_t)
    out_ref[...] = rsn * er / srn + (1.0 - rsn) * es / ssn


def _final_combine(part, xs_row, ni_col, ni_row, rel_row):
    n_pad = xs_row.shape[1]
    b = rel_row.shape[1]
    return pl.pallas_call(
        _final_body,
        grid=(1,),
        in_specs=[
            pl.BlockSpec((2, n_pad), lambda i: (0, 0)),
            pl.BlockSpec((1, n_pad), lambda i: (0, 0)),
            pl.BlockSpec((n_pad, 1), lambda i: (0, 0)),
            pl.BlockSpec((1, n_pad), lambda i: (0, 0)),
            pl.BlockSpec((1, b), lambda i: (0, 0)),
        ],
        out_specs=pl.BlockSpec((1, n_pad), lambda i: (0, 0)),
        out_shape=jax.ShapeDtypeStruct((1, n_pad), jnp.float32),
    )(part, xs_row, ni_col, ni_row, rel_row)


def kernel(node_attrs, edge_attrs, instruction_batch, distribution,
           node_prop_similarities, relation_similarity,
           W_props, W_edge, w_node_score, w_rel_score,
           edge_indices, node_indices, edge_batch_indices):
    n, _, h = node_attrs.shape
    e = edge_attrs.shape[0]
    b = instruction_batch.shape[0]
    n_pad = -(-n // (NS * LANES * 2)) * (NS * LANES * 2)

    s_edge, x_state = _all_scores(edge_attrs, node_attrs, instruction_batch,
                                  node_prop_similarities, W_edge, W_props,
                                  w_rel_score, w_node_score,
                                  edge_batch_indices, node_indices)

    zeros = jnp.zeros((n_pad,), jnp.float32)
    part = _sc_scatter(s_edge.reshape(e), edge_indices[0], edge_indices[1],
                       distribution, zeros, n_pad)

    xs_row = jnp.pad(x_state.reshape(1, n), ((0, 0), (0, n_pad - n)))
    ni_pad = jnp.pad(node_indices, (0, n_pad - n), constant_values=b)
    out = _final_combine(part, xs_row, ni_pad.reshape(n_pad, 1),
                         ni_pad.reshape(1, n_pad),
                         relation_similarity.reshape(1, b))
    return out.reshape(n_pad)[:n]


# speedup vs baseline: 1.7615x; 1.7615x over previous
import jax
import jax.numpy as jnp
from jax import lax
from jax.experimental import pallas as pl
from jax.experimental.pallas import tpu as pltpu
from jax.experimental.pallas import tpu_sc as plsc

NC = 2
NS = 16
LANES = 16


def _split_dot(v, m):
    hi = v.astype(jnp.bfloat16).astype(jnp.float32)
    return (jnp.dot(hi, m, preferred_element_type=jnp.float32)
            + jnp.dot(v - hi, m, preferred_element_type=jnp.float32))


def _gather_dot(onehot, tbl):
    hi = tbl.astype(jnp.bfloat16).astype(jnp.float32)
    return (jnp.dot(onehot, hi, preferred_element_type=jnp.float32)
            + jnp.dot(onehot, tbl - hi, preferred_element_type=jnp.float32))


def _scores_body(ebi_ref, ea_ref, ni_ref, na_ref, instr_ref, nps_ref,
                 wedge_ref, wprops_ref, wrel_ref, wnode_ref,
                 s_ref, x_ref):
    be = ea_ref.shape[0]
    b, h = instr_ref.shape
    p_cnt = wprops_ref.shape[0]
    bn = na_ref.shape[0]

    ebi_t = jnp.swapaxes(ebi_ref[0], 0, 1)
    iota_b = lax.broadcasted_iota(jnp.int32, (128, b), 1)
    chunks = [(ebi_t[:, q:q + 1] == iota_b).astype(jnp.float32)
              for q in range(be // 128)]
    onehot_e = jnp.concatenate(chunks, axis=0)
    instr_g = jnp.dot(onehot_e, instr_ref[...],
                      preferred_element_type=jnp.float32)
    y = jnp.dot(instr_g * ea_ref[...], wedge_ref[...],
                preferred_element_type=jnp.float32)
    y = jnp.where(y > 0, y, jnp.exp(y) - 1.0)
    s_ref[...] = jnp.sum(y * wrel_ref[...], axis=1).reshape(1, be // 128, 128)

    @pl.when(pl.program_id(0) % NODE_EVERY == 0)
    def _node():
        onehot = (ni_ref[...] == lax.broadcasted_iota(jnp.int32, (bn, b), 1))
        onehot = onehot.astype(jnp.float32)
        ib = _gather_dot(onehot, instr_ref[...])
        aps = _gather_dot(onehot, nps_ref[...])
        acc = jnp.zeros((bn, h), jnp.float32)
        for p in range(p_cnt):
            zp = aps[:, p:p + 1] * ib * na_ref[:, p, :]
            acc = acc + jnp.dot(zp, wprops_ref[p],
                                preferred_element_type=jnp.float32)
        yk = jnp.where(acc > 0, acc, jnp.exp(acc) - 1.0)
        x_ref[...] = jnp.sum(yk * wnode_ref[...], axis=1).reshape(1, 1, bn)


NODE_EVERY = 10


def _all_scores(edge_attrs, node_attrs, instruction_batch,
                node_prop_similarities, w_edge, w_props, w_rel, w_node,
                ebi, ni):
    e, h = edge_attrs.shape
    n, p_cnt, _ = node_attrs.shape
    b = instruction_batch.shape[0]
    be = 3200
    grid = e // be
    bn = n // (grid // NODE_EVERY)
    return pl.pallas_call(
        _scores_body,
        grid=(grid,),
        in_specs=[
            pl.BlockSpec((1, be // 128, 128), lambda i: (i, 0, 0)),
            pl.BlockSpec((be, h), lambda i: (i, 0)),
            pl.BlockSpec((bn, 1), lambda i: (i // NODE_EVERY, 0)),
            pl.BlockSpec((bn, p_cnt, h), lambda i: (i // NODE_EVERY, 0, 0)),
            pl.BlockSpec((b, h), lambda i: (0, 0)),
            pl.BlockSpec((b, p_cnt), lambda i: (0, 0)),
            pl.BlockSpec((h, h), lambda i: (0, 0)),
            pl.BlockSpec((p_cnt, h, h), lambda i: (0, 0, 0)),
            pl.BlockSpec((1, h), lambda i: (0, 0)),
            pl.BlockSpec((1, h), lambda i: (0, 0)),
        ],
        out_specs=[
            pl.BlockSpec((1, be // 128, 128), lambda i: (i, 0, 0)),
            pl.BlockSpec((1, 1, bn), lambda i: (i // NODE_EVERY, 0, 0)),
        ],
        out_shape=[
            jax.ShapeDtypeStruct((grid, be // 128, 128), jnp.float32),
            jax.ShapeDtypeStruct((grid // NODE_EVERY, 1, bn), jnp.float32),
        ],
    )(ebi.reshape(grid, be // 128, 128), edge_attrs, ni.reshape(n, 1), node_attrs,
      instruction_batch, node_prop_similarities, w_edge, w_props,
      w_rel.reshape(1, h), w_node.reshape(1, h))


def _sc_scatter_body(n_pad, ec, s_hbm, src_hbm, dst_hbm, dist_hbm, zeros_hbm,
                     part_hbm, s_v, src_v, dst_v, dist_v, acc_v, tmp_v, red_v,
                     kbuf_v, shared_acc, sem):
    c = lax.axis_index("c")
    s = lax.axis_index("s")
    wid = c * NS + s
    ebase = wid * ec
    nslice = n_pad // NS

    cps = [
        pltpu.async_copy(s_hbm.at[pl.ds(ebase, ec)], s_v, sem),
        pltpu.async_copy(src_hbm.at[pl.ds(ebase, ec)], src_v, sem),
        pltpu.async_copy(dst_hbm.at[pl.ds(ebase, ec)], dst_v, sem),
        pltpu.async_copy(dist_hbm, dist_v, sem),
        pltpu.async_copy(zeros_hbm, acc_v, sem),
    ]
    for cp in cps:
        cp.wait()

    lanes_iota = lax.iota(jnp.int32, LANES)
    kbuf_v[pl.ds(LANES, LANES)] = jnp.full((LANES,), -1, jnp.int32)

    @pl.loop(0, ec // LANES, unroll=4)
    def _edges(i):
        sl = pl.ds(i * LANES, LANES)
        d16 = plsc.load_gather(dist_v, [src_v[sl]])
        k, v = plsc.sort_key_val(dst_v[sl], d16 * s_v[sl])
        cs = plsc.cumsum(v)
        kbuf_v[pl.ds(0, LANES)] = k
        k_next = plsc.load_gather(kbuf_v, [lanes_iota + 1])
        last = k != k_next
        plsc.addupdate_scatter(acc_v, [k], cs, mask=last)
        plsc.addupdate_scatter(acc_v, [k_next], -cs,
                               mask=last & (lanes_iota < LANES - 1))

    pltpu.sync_copy(acc_v, shared_acc.at[s])
    plsc.subcore_barrier()
    nbase = s * nslice
    pltpu.sync_copy(shared_acc.at[:, pl.ds(nbase, nslice)], tmp_v)

    @pl.loop(0, nslice // LANES, unroll=4)
    def _acc(j):
        sl = pl.ds(j * LANES, LANES)
        tot = tmp_v[0, sl]
        for t in range(1, NS):
            tot = tot + tmp_v[t, sl]
        red_v[sl] = tot

    pltpu.sync_copy(red_v, part_hbm.at[c, pl.ds(nbase, nslice)])


def _sc_scatter(s_flat, src, dst, dist, zeros, n_pad):
    e = s_flat.shape[0]
    n = dist.shape[0]
    ec = e // (NC * NS)
    nslice = n_pad // NS
    mesh = plsc.VectorSubcoreMesh(core_axis_name="c", subcore_axis_name="s",
                                  num_cores=NC, num_subcores=NS)
    import functools
    body = functools.partial(_sc_scatter_body, n_pad, ec)
    return pl.kernel(
        body,
        out_type=jax.ShapeDtypeStruct((NC, n_pad), jnp.float32),
        mesh=mesh,
        compiler_params=pltpu.CompilerParams(needs_layout_passes=False),
        scratch_types=[
            pltpu.VMEM((ec,), jnp.float32),
            pltpu.VMEM((ec,), jnp.int32),
            pltpu.VMEM((ec,), jnp.int32),
            pltpu.VMEM((n,), jnp.float32),
            pltpu.VMEM((n_pad,), jnp.float32),
            pltpu.VMEM((NS, nslice), jnp.float32),
            pltpu.VMEM((nslice,), jnp.float32),
            pltpu.VMEM((2 * LANES,), jnp.int32),
            pltpu.VMEM_SHARED((NS, n_pad), jnp.float32),
            pltpu.SemaphoreType.DMA,
        ],
    )(s_flat, src, dst, dist, zeros)


def _final_body(part_ref, xs_ref, nic_ref, nir_ref, rel_ref, out_ref):
    n_pad = part_ref.shape[1]
    b = rel_ref.shape[1]
    red = part_ref[0:1, :] + part_ref[1:2, :]
    er = jnp.exp(red)
    es = jnp.exp(xs_ref[...])
    nic = nic_ref[...]
    nir = nir_ref[...]
    onehot = (nic == lax.broadcasted_iota(jnp.int32, (n_pad, b), 1))
    onehot = onehot.astype(jnp.float32)
    onehot_t = (nir == lax.broadcasted_iota(jnp.int32, (b, n_pad), 0))
    onehot_t = onehot_t.astype(jnp.float32)
    s_r = _split_dot(er, onehot)
    s_s = _split_dot(es, onehot)
    srn = jnp.maximum(_split_dot(s_r, onehot_t), 1e-30)
    ssn = jnp.maximum(_split_dot(s_s, onehot_t), 1e-30)
    rsn = _split_dot(rel_ref[...], onehot_t)
    out_ref[...] = rsn * er / srn + (1.0 - rsn) * es / ssn


def _final_combine(part, xs_row, ni_pad, rel_row):
    n_pad = xs_row.shape[1]
    b = rel_row.shape[1]
    return pl.pallas_call(
        _final_body,
        grid=(1,),
        in_specs=[
            pl.BlockSpec((2, n_pad), lambda i: (0, 0)),
            pl.BlockSpec((1, n_pad), lambda i: (0, 0)),
            pl.BlockSpec((n_pad, 1), lambda i: (0, 0)),
            pl.BlockSpec((1, n_pad), lambda i: (0, 0)),
            pl.BlockSpec((1, b), lambda i: (0, 0)),
        ],
        out_specs=pl.BlockSpec((1, n_pad), lambda i: (0, 0)),
        out_shape=jax.ShapeDtypeStruct((1, n_pad), jnp.float32),
    )(part, xs_row, ni_pad.reshape(n_pad, 1), ni_pad.reshape(1, n_pad),
      rel_row)


def kernel(node_attrs, edge_attrs, instruction_batch, distribution,
           node_prop_similarities, relation_similarity,
           W_props, W_edge, w_node_score, w_rel_score,
           edge_indices, node_indices, edge_batch_indices):
    n, _, h = node_attrs.shape
    e = edge_attrs.shape[0]
    b = instruction_batch.shape[0]
    n_pad = -(-n // (NS * LANES * 2)) * (NS * LANES * 2)

    s_edge, x_state = _all_scores(edge_attrs, node_attrs, instruction_batch,
                                  node_prop_similarities, W_edge, W_props,
                                  w_rel_score, w_node_score,
                                  edge_batch_indices, node_indices)

    zeros = jnp.zeros((n_pad,), jnp.float32)
    part = _sc_scatter(s_edge.reshape(e), edge_indices[0], edge_indices[1],
                       distribution, zeros, n_pad)

    xs_row = jnp.pad(x_state.reshape(1, n), ((0, 0), (0, n_pad - n)))
    ni_pad = jnp.pad(node_indices, (0, n_pad - n), constant_values=b)
    out = _final_combine(part, xs_row, ni_pad,
                         relation_similarity.reshape(1, b))
    return out.reshape(n_pad)[:n]


# speedup vs baseline: 2.0062x; 1.1389x over previous
import jax
import jax.numpy as jnp
from jax import lax
from jax.experimental import pallas as pl
from jax.experimental.pallas import tpu as pltpu
from jax.experimental.pallas import tpu_sc as plsc

NC = 2
NS = 16
LANES = 16


def _split_dot(v, m):
    hi = v.astype(jnp.bfloat16).astype(jnp.float32)
    return (jnp.dot(hi, m, preferred_element_type=jnp.float32)
            + jnp.dot(v - hi, m, preferred_element_type=jnp.float32))


def _gather_dot(onehot, tbl):
    hi = tbl.astype(jnp.bfloat16).astype(jnp.float32)
    return (jnp.dot(onehot, hi, preferred_element_type=jnp.float32)
            + jnp.dot(onehot, tbl - hi, preferred_element_type=jnp.float32))


def _edge_body(ebi_ref, ea_ref, instr_ref, wedge_ref, wrel_ref, s_ref):
    be = ea_ref.shape[0]
    b = instr_ref.shape[0]
    ebi_t = jnp.swapaxes(ebi_ref[0], 0, 1)
    iota_b = lax.broadcasted_iota(jnp.int32, (128, b), 1)
    chunks = [(ebi_t[:, q:q + 1] == iota_b).astype(jnp.float32)
              for q in range(be // 128)]
    onehot_e = jnp.concatenate(chunks, axis=0)
    instr_g = jnp.dot(onehot_e, instr_ref[...],
                      preferred_element_type=jnp.float32)
    y = jnp.dot(instr_g * ea_ref[...], wedge_ref[...],
                preferred_element_type=jnp.float32)
    y = jnp.where(y > 0, y, jnp.exp(y) - 1.0)
    s_ref[...] = jnp.sum(y * wrel_ref[...], axis=1).reshape(1, be // 128, 128)


def _edge_scores(edge_attrs, instruction_batch, w_edge, w_rel, ebi):
    e, h = edge_attrs.shape
    b = instruction_batch.shape[0]
    be = 3200
    grid = e // be
    return pl.pallas_call(
        _edge_body,
        grid=(grid,),
        in_specs=[
            pl.BlockSpec((1, be // 128, 128), lambda i: (i, 0, 0)),
            pl.BlockSpec((be, h), lambda i: (i, 0)),
            pl.BlockSpec((b, h), lambda i: (0, 0)),
            pl.BlockSpec((h, h), lambda i: (0, 0)),
            pl.BlockSpec((1, h), lambda i: (0, 0)),
        ],
        out_specs=pl.BlockSpec((1, be // 128, 128), lambda i: (i, 0, 0)),
        out_shape=jax.ShapeDtypeStruct((grid, be // 128, 128), jnp.float32),
    )(ebi.reshape(grid, be // 128, 128), edge_attrs, instruction_batch,
      w_edge, w_rel.reshape(1, h))


def _node_body(ni_ref, na_ref, instr_ref, nps_ref, wprops_ref, wnode_ref,
               x_ref):
    bn = na_ref.shape[0]
    b, h = instr_ref.shape
    p_cnt = wprops_ref.shape[0]
    onehot = (ni_ref[...] == lax.broadcasted_iota(jnp.int32, (bn, b), 1))
    onehot = onehot.astype(jnp.float32)
    ib = _gather_dot(onehot, instr_ref[...])
    aps = _gather_dot(onehot, nps_ref[...])
    acc = jnp.zeros((bn, h), jnp.float32)
    for p in range(p_cnt):
        zp = aps[:, p:p + 1] * ib * na_ref[:, p, :]
        acc = acc + jnp.dot(zp, wprops_ref[p],
                            preferred_element_type=jnp.float32)
    yk = jnp.where(acc > 0, acc, jnp.exp(acc) - 1.0)
    x_ref[...] = jnp.sum(yk * wnode_ref[...], axis=1).reshape(1, 1, bn)


def _node_scores(node_attrs, instruction_batch, node_prop_similarities,
                 w_props, w_node, ni):
    n, p_cnt, h = node_attrs.shape
    b = instruction_batch.shape[0]
    bn = 1000
    grid = n // bn
    return pl.pallas_call(
        _node_body,
        grid=(grid,),
        in_specs=[
            pl.BlockSpec((bn, 1), lambda i: (i, 0)),
            pl.BlockSpec((bn, p_cnt, h), lambda i: (i, 0, 0)),
            pl.BlockSpec((b, h), lambda i: (0, 0)),
            pl.BlockSpec((b, p_cnt), lambda i: (0, 0)),
            pl.BlockSpec((p_cnt, h, h), lambda i: (0, 0, 0)),
            pl.BlockSpec((1, h), lambda i: (0, 0)),
        ],
        out_specs=pl.BlockSpec((1, 1, bn), lambda i: (i, 0, 0)),
        out_shape=jax.ShapeDtypeStruct((grid, 1, bn), jnp.float32),
    )(ni.reshape(n, 1), node_attrs, instruction_batch,
      node_prop_similarities, w_props, w_node.reshape(1, h))


def _sc_scatter_body(n_pad, ec, s_hbm, src_hbm, dst_hbm, dist_hbm, zeros_hbm,
                     part_hbm, s_v, src_v, dst_v, dist_v, acc_v, tmp_v, red_v,
                     kbuf_v, shared_acc, sem):
    c = lax.axis_index("c")
    s = lax.axis_index("s")
    wid = c * NS + s
    ebase = wid * ec
    nslice = n_pad // NS

    cps = [
        pltpu.async_copy(s_hbm.at[pl.ds(ebase, ec)], s_v, sem),
        pltpu.async_copy(src_hbm.at[pl.ds(ebase, ec)], src_v, sem),
        pltpu.async_copy(dst_hbm.at[pl.ds(ebase, ec)], dst_v, sem),
        pltpu.async_copy(dist_hbm, dist_v, sem),
        pltpu.async_copy(zeros_hbm, acc_v, sem),
    ]
    for cp in cps:
        cp.wait()

    lanes_iota = lax.iota(jnp.int32, LANES)
    kbuf_v[pl.ds(LANES, LANES)] = jnp.full((LANES,), -1, jnp.int32)

    @pl.loop(0, ec // LANES, unroll=4)
    def _edges(i):
        sl = pl.ds(i * LANES, LANES)
        d16 = plsc.load_gather(dist_v, [src_v[sl]])
        k, v = plsc.sort_key_val(dst_v[sl], d16 * s_v[sl])
        cs = plsc.cumsum(v)
        kbuf_v[pl.ds(0, LANES)] = k
        k_next = plsc.load_gather(kbuf_v, [lanes_iota + 1])
        last = k != k_next
        plsc.addupdate_scatter(acc_v, [k], cs, mask=last)
        plsc.addupdate_scatter(acc_v, [k_next], -cs,
                               mask=last & (lanes_iota < LANES - 1))

    pltpu.sync_copy(acc_v, shared_acc.at[s])
    plsc.subcore_barrier()
    nbase = s * nslice
    pltpu.sync_copy(shared_acc.at[:, pl.ds(nbase, nslice)], tmp_v)

    @pl.loop(0, nslice // LANES, unroll=4)
    def _acc(j):
        sl = pl.ds(j * LANES, LANES)
        tot = tmp_v[0, sl]
        for t in range(1, NS):
            tot = tot + tmp_v[t, sl]
        red_v[sl] = tot

    pltpu.sync_copy(red_v, part_hbm.at[c, pl.ds(nbase, nslice)])


def _sc_scatter(s_flat, src, dst, dist, zeros, n_pad):
    e = s_flat.shape[0]
    n = dist.shape[0]
    ec = e // (NC * NS)
    nslice = n_pad // NS
    mesh = plsc.VectorSubcoreMesh(core_axis_name="c", subcore_axis_name="s",
                                  num_cores=NC, num_subcores=NS)
    import functools
    body = functools.partial(_sc_scatter_body, n_pad, ec)
    return pl.kernel(
        body,
        out_type=jax.ShapeDtypeStruct((NC, n_pad), jnp.float32),
        mesh=mesh,
        compiler_params=pltpu.CompilerParams(needs_layout_passes=False),
        scratch_types=[
            pltpu.VMEM((ec,), jnp.float32),
            pltpu.VMEM((ec,), jnp.int32),
            pltpu.VMEM((ec,), jnp.int32),
            pltpu.VMEM((n,), jnp.float32),
            pltpu.VMEM((n_pad,), jnp.float32),
            pltpu.VMEM((NS, nslice), jnp.float32),
            pltpu.VMEM((nslice,), jnp.float32),
            pltpu.VMEM((2 * LANES,), jnp.int32),
            pltpu.VMEM_SHARED((NS, n_pad), jnp.float32),
            pltpu.SemaphoreType.DMA,
        ],
    )(s_flat, src, dst, dist, zeros)


def _final_body(part_ref, xs_ref, nic_ref, nir_ref, rel_ref, out_ref):
    n_pad = part_ref.shape[1]
    b = rel_ref.shape[1]
    red = part_ref[0:1, :] + part_ref[1:2, :]
    er = jnp.exp(red)
    es = jnp.exp(xs_ref[...])
    nic = nic_ref[...]
    nir = nir_ref[...]
    onehot = (nic == lax.broadcasted_iota(jnp.int32, (n_pad, b), 1))
    onehot = onehot.astype(jnp.float32)
    onehot_t = (nir == lax.broadcasted_iota(jnp.int32, (b, n_pad), 0))
    onehot_t = onehot_t.astype(jnp.float32)
    s_r = _split_dot(er, onehot)
    s_s = _split_dot(es, onehot)
    srn = jnp.maximum(_split_dot(s_r, onehot_t), 1e-30)
    ssn = jnp.maximum(_split_dot(s_s, onehot_t), 1e-30)
    rsn = _split_dot(rel_ref[...], onehot_t)
    out_ref[...] = rsn * er / srn + (1.0 - rsn) * es / ssn


def _final_combine(part, xs_row, ni_pad, rel_row):
    n_pad = xs_row.shape[1]
    b = rel_row.shape[1]
    return pl.pallas_call(
        _final_body,
        grid=(1,),
        in_specs=[
            pl.BlockSpec((2, n_pad), lambda i: (0, 0)),
            pl.BlockSpec((1, n_pad), lambda i: (0, 0)),
            pl.BlockSpec((n_pad, 1), lambda i: (0, 0)),
            pl.BlockSpec((1, n_pad), lambda i: (0, 0)),
            pl.BlockSpec((1, b), lambda i: (0, 0)),
        ],
        out_specs=pl.BlockSpec((1, n_pad), lambda i: (0, 0)),
        out_shape=jax.ShapeDtypeStruct((1, n_pad), jnp.float32),
    )(part, xs_row, ni_pad.reshape(n_pad, 1), ni_pad.reshape(1, n_pad),
      rel_row)


def kernel(node_attrs, edge_attrs, instruction_batch, distribution,
           node_prop_similarities, relation_similarity,
           W_props, W_edge, w_node_score, w_rel_score,
           edge_indices, node_indices, edge_batch_indices):
    n, _, h = node_attrs.shape
    e = edge_attrs.shape[0]
    b = instruction_batch.shape[0]
    n_pad = -(-n // (NS * LANES * 2)) * (NS * LANES * 2)

    s_edge = _edge_scores(edge_attrs, instruction_batch, W_edge,
                          w_rel_score, edge_batch_indices)
    x_state = _node_scores(node_attrs, instruction_batch,
                           node_prop_similarities, W_props, w_node_score,
                           node_indices)

    zeros = jnp.zeros((n_pad,), jnp.float32)
    part = _sc_scatter(s_edge.reshape(e), edge_indices[0], edge_indices[1],
                       distribution, zeros, n_pad)

    xs_row = jnp.pad(x_state.reshape(1, n), ((0, 0), (0, n_pad - n)))
    ni_pad = jnp.pad(node_indices, (0, n_pad - n), constant_values=b)
    out = _final_combine(part, xs_row, ni_pad,
                         relation_similarity.reshape(1, b))
    return out.reshape(n_pad)[:n]


# speedup vs baseline: 2.2556x; 1.1243x over previous
import jax
import jax.numpy as jnp
from jax import lax
from jax.experimental import pallas as pl
from jax.experimental.pallas import tpu as pltpu
from jax.experimental.pallas import tpu_sc as plsc

NC = 2
NS = 16
LANES = 16


def _split_dot(v, m):
    hi = v.astype(jnp.bfloat16).astype(jnp.float32)
    return (jnp.dot(hi, m, preferred_element_type=jnp.float32)
            + jnp.dot(v - hi, m, preferred_element_type=jnp.float32))


def _gather_dot(onehot, tbl):
    hi = tbl.astype(jnp.bfloat16).astype(jnp.float32)
    return (jnp.dot(onehot, hi, preferred_element_type=jnp.float32)
            + jnp.dot(onehot, tbl - hi, preferred_element_type=jnp.float32))


def _edge_body(ebi_ref, ea_ref, instr_ref, wedge_ref, wrel_ref, s_ref):
    be = ea_ref.shape[0]
    b = instr_ref.shape[0]
    ebi_t = jnp.swapaxes(ebi_ref[0], 0, 1)
    iota_b = lax.broadcasted_iota(jnp.int32, (128, b), 1)
    chunks = [(ebi_t[:, q:q + 1] == iota_b).astype(jnp.float32)
              for q in range(be // 128)]
    onehot_e = jnp.concatenate(chunks, axis=0)
    instr_g = jnp.dot(onehot_e, instr_ref[...],
                      preferred_element_type=jnp.float32)
    y = jnp.dot(instr_g * ea_ref[...], wedge_ref[...],
                preferred_element_type=jnp.float32)
    y = jnp.where(y > 0, y, jnp.exp(y) - 1.0)
    s_ref[...] = jnp.sum(y * wrel_ref[...], axis=1).reshape(1, be // 128, 128)


def _edge_scores(edge_attrs, instruction_batch, w_edge, w_rel, ebi):
    e, h = edge_attrs.shape
    b = instruction_batch.shape[0]
    be = 6400
    grid = e // be
    return pl.pallas_call(
        _edge_body,
        grid=(grid,),
        in_specs=[
            pl.BlockSpec((1, be // 128, 128), lambda i: (i, 0, 0)),
            pl.BlockSpec((be, h), lambda i: (i, 0)),
            pl.BlockSpec((b, h), lambda i: (0, 0)),
            pl.BlockSpec((h, h), lambda i: (0, 0)),
            pl.BlockSpec((1, h), lambda i: (0, 0)),
        ],
        out_specs=pl.BlockSpec((1, be // 128, 128), lambda i: (i, 0, 0)),
        out_shape=jax.ShapeDtypeStruct((grid, be // 128, 128), jnp.float32),
    )(ebi.reshape(grid, be // 128, 128), edge_attrs, instruction_batch,
      w_edge, w_rel.reshape(1, h))


def _node_body(ni_ref, na_ref, instr_ref, nps_ref, wprops_ref, wnode_ref,
               x_ref):
    bn = na_ref.shape[0]
    b, h = instr_ref.shape
    p_cnt = wprops_ref.shape[0]
    onehot = (ni_ref[...] == lax.broadcasted_iota(jnp.int32, (bn, b), 1))
    onehot = onehot.astype(jnp.float32)
    ib = _gather_dot(onehot, instr_ref[...])
    aps = _gather_dot(onehot, nps_ref[...])
    acc = jnp.zeros((bn, h), jnp.float32)
    for p in range(p_cnt):
        zp = aps[:, p:p + 1] * ib * na_ref[:, p, :]
        acc = acc + jnp.dot(zp, wprops_ref[p],
                            preferred_element_type=jnp.float32)
    yk = jnp.where(acc > 0, acc, jnp.exp(acc) - 1.0)
    x_ref[...] = jnp.sum(yk * wnode_ref[...], axis=1).reshape(1, 1, bn)


def _node_scores(node_attrs, instruction_batch, node_prop_similarities,
                 w_props, w_node, ni):
    n, p_cnt, h = node_attrs.shape
    b = instruction_batch.shape[0]
    bn = 1000
    grid = n // bn
    return pl.pallas_call(
        _node_body,
        grid=(grid,),
        in_specs=[
            pl.BlockSpec((bn, 1), lambda i: (i, 0)),
            pl.BlockSpec((bn, p_cnt, h), lambda i: (i, 0, 0)),
            pl.BlockSpec((b, h), lambda i: (0, 0)),
            pl.BlockSpec((b, p_cnt), lambda i: (0, 0)),
            pl.BlockSpec((p_cnt, h, h), lambda i: (0, 0, 0)),
            pl.BlockSpec((1, h), lambda i: (0, 0)),
        ],
        out_specs=pl.BlockSpec((1, 1, bn), lambda i: (i, 0, 0)),
        out_shape=jax.ShapeDtypeStruct((grid, 1, bn), jnp.float32),
    )(ni.reshape(n, 1), node_attrs, instruction_batch,
      node_prop_similarities, w_props, w_node.reshape(1, h))


def _sc_scatter_body(n_pad, ec, s_hbm, src_hbm, dst_hbm, dist_hbm, zeros_hbm,
                     part_hbm, s_v, src_v, dst_v, dist_v, acc_v, tmp_v, red_v,
                     kbuf_v, shared_acc, sem):
    c = lax.axis_index("c")
    s = lax.axis_index("s")
    wid = c * NS + s
    ebase = wid * ec
    nslice = n_pad // NS

    cps = [
        pltpu.async_copy(s_hbm.at[pl.ds(ebase, ec)], s_v, sem),
        pltpu.async_copy(src_hbm.at[pl.ds(ebase, ec)], src_v, sem),
        pltpu.async_copy(dst_hbm.at[pl.ds(ebase, ec)], dst_v, sem),
        pltpu.async_copy(dist_hbm, dist_v, sem),
        pltpu.async_copy(zeros_hbm, acc_v, sem),
    ]
    for cp in cps:
        cp.wait()

    lanes_iota = lax.iota(jnp.int32, LANES)
    kbuf_v[pl.ds(LANES, LANES)] = jnp.full((LANES,), -1, jnp.int32)

    @pl.loop(0, ec // LANES, unroll=4)
    def _edges(i):
        sl = pl.ds(i * LANES, LANES)
        d16 = plsc.load_gather(dist_v, [src_v[sl]])
        k, v = plsc.sort_key_val(dst_v[sl], d16 * s_v[sl])
        cs = plsc.cumsum(v)
        kbuf_v[pl.ds(0, LANES)] = k
        k_next = plsc.load_gather(kbuf_v, [lanes_iota + 1])
        last = k != k_next
        plsc.addupdate_scatter(acc_v, [k], cs, mask=last)
        plsc.addupdate_scatter(acc_v, [k_next], -cs,
                               mask=last & (lanes_iota < LANES - 1))

    pltpu.sync_copy(acc_v, shared_acc.at[s])
    plsc.subcore_barrier()
    nbase = s * nslice
    pltpu.sync_copy(shared_acc.at[:, pl.ds(nbase, nslice)], tmp_v)

    @pl.loop(0, nslice // LANES, unroll=4)
    def _acc(j):
        sl = pl.ds(j * LANES, LANES)
        tot = tmp_v[0, sl]
        for t in range(1, NS):
            tot = tot + tmp_v[t, sl]
        red_v[sl] = tot

    pltpu.sync_copy(red_v, part_hbm.at[c, pl.ds(nbase, nslice)])


def _sc_scatter(s_flat, src, dst, dist, zeros, n_pad):
    e = s_flat.shape[0]
    n = dist.shape[0]
    ec = e // (NC * NS)
    nslice = n_pad // NS
    mesh = plsc.VectorSubcoreMesh(core_axis_name="c", subcore_axis_name="s",
                                  num_cores=NC, num_subcores=NS)
    import functools
    body = functools.partial(_sc_scatter_body, n_pad, ec)
    return pl.kernel(
        body,
        out_type=jax.ShapeDtypeStruct((NC, n_pad), jnp.float32),
        mesh=mesh,
        compiler_params=pltpu.CompilerParams(needs_layout_passes=False),
        scratch_types=[
            pltpu.VMEM((ec,), jnp.float32),
            pltpu.VMEM((ec,), jnp.int32),
            pltpu.VMEM((ec,), jnp.int32),
            pltpu.VMEM((n,), jnp.float32),
            pltpu.VMEM((n_pad,), jnp.float32),
            pltpu.VMEM((NS, nslice), jnp.float32),
            pltpu.VMEM((nslice,), jnp.float32),
            pltpu.VMEM((2 * LANES,), jnp.int32),
            pltpu.VMEM_SHARED((NS, n_pad), jnp.float32),
            pltpu.SemaphoreType.DMA,
        ],
    )(s_flat, src, dst, dist, zeros)


def _final_body(part_ref, xs_ref, nic_ref, nir_ref, rel_ref, out_ref):
    n_pad = part_ref.shape[1]
    b = rel_ref.shape[1]
    red = part_ref[0:1, :] + part_ref[1:2, :]
    er = jnp.exp(red)
    es = jnp.exp(xs_ref[...])
    nic = nic_ref[...]
    nir = nir_ref[...]
    onehot = (nic == lax.broadcasted_iota(jnp.int32, (n_pad, b), 1))
    onehot = onehot.astype(jnp.float32)
    onehot_t = (nir == lax.broadcasted_iota(jnp.int32, (b, n_pad), 0))
    onehot_t = onehot_t.astype(jnp.float32)
    s_r = _split_dot(er, onehot)
    s_s = _split_dot(es, onehot)
    srn = jnp.maximum(_split_dot(s_r, onehot_t), 1e-30)
    ssn = jnp.maximum(_split_dot(s_s, onehot_t), 1e-30)
    rsn = _split_dot(rel_ref[...], onehot_t)
    out_ref[...] = rsn * er / srn + (1.0 - rsn) * es / ssn


def _final_combine(part, xs_row, ni_pad, rel_row):
    n_pad = xs_row.shape[1]
    b = rel_row.shape[1]
    return pl.pallas_call(
        _final_body,
        grid=(1,),
        in_specs=[
            pl.BlockSpec((2, n_pad), lambda i: (0, 0)),
            pl.BlockSpec((1, n_pad), lambda i: (0, 0)),
            pl.BlockSpec((n_pad, 1), lambda i: (0, 0)),
            pl.BlockSpec((1, n_pad), lambda i: (0, 0)),
            pl.BlockSpec((1, b), lambda i: (0, 0)),
        ],
        out_specs=pl.BlockSpec((1, n_pad), lambda i: (0, 0)),
        out_shape=jax.ShapeDtypeStruct((1, n_pad), jnp.float32),
    )(part, xs_row, ni_pad.reshape(n_pad, 1), ni_pad.reshape(1, n_pad),
      rel_row)


def kernel(node_attrs, edge_attrs, instruction_batch, distribution,
           node_prop_similarities, relation_similarity,
           W_props, W_edge, w_node_score, w_rel_score,
           edge_indices, node_indices, edge_batch_indices):
    n, _, h = node_attrs.shape
    e = edge_attrs.shape[0]
    b = instruction_batch.shape[0]
    n_pad = -(-n // (NS * LANES * 2)) * (NS * LANES * 2)

    s_edge = _edge_scores(edge_attrs, instruction_batch, W_edge,
                          w_rel_score, edge_batch_indices)
    x_state = _node_scores(node_attrs, instruction_batch,
                           node_prop_similarities, W_props, w_node_score,
                           node_indices)

    zeros = jnp.zeros((n_pad,), jnp.float32)
    part = _sc_scatter(s_edge.reshape(e), edge_indices[0], edge_indices[1],
                       distribution, zeros, n_pad)

    xs_row = jnp.pad(x_state.reshape(1, n), ((0, 0), (0, n_pad - n)))
    ni_pad = jnp.pad(node_indices, (0, n_pad - n), constant_values=b)
    out = _final_combine(part, xs_row, ni_pad,
                         relation_similarity.reshape(1, b))
    return out.reshape(n_pad)[:n]
